# Initial kernel scaffold; baseline (speedup 1.0000x reference)
import jax
import jax.numpy as jnp
from jax.experimental import pallas as pl


def kernel(hidden_states, gate_w, w_gate, w_up, w_down):
    raise NotImplementedError("write your pallas kernel here")


# speedup vs baseline: 5.7708x; 5.7708x over previous
import functools

import jax
import jax.numpy as jnp
from jax import lax
from jax.experimental import pallas as pl
from jax.experimental.pallas import tpu as pltpu
from jax.experimental.pallas import tpu_sc as plsc

S = 2048
H = 1024
I = 768
E = 64
T = 32
NT = 128
PAD = NT * T

NW = 32
RW = S // NW


def _router_body(x_ref, gw_ref, logits_ref, pos_ref, te_ref):
    x = x_ref[...]
    gw = gw_ref[...]
    logits = lax.dot_general(
        x, gw, (((1,), (1,)), ((), ())),
        preferred_element_type=jnp.float32,
        precision=lax.Precision.DEFAULT,
    )
    logits_ref[...] = logits

    m = jnp.max(logits, axis=1, keepdims=True)
    eiota = lax.broadcasted_iota(jnp.int32, (S, E), 1)
    sel = jnp.min(jnp.where(logits == m, eiota, E), axis=1)

    onehot = (eiota == sel[:, None]).astype(jnp.int32)

    c = onehot
    k = 1
    while k < S:
        c = c + jnp.concatenate(
            [jnp.zeros((k, E), jnp.int32), c[: S - k, :]], axis=0)
        k *= 2

    rank = jnp.sum(onehot * c, axis=1) - 1
    counts = c[S - 1 : S, :]
    padded = ((counts + (T - 1)) // T) * T

    pc = padded
    k = 1
    while k < E:
        pc = pc + jnp.concatenate(
            [jnp.zeros((1, k), jnp.int32), pc[:, : E - k]], axis=1)
        k *= 2
    off_excl = pc - padded
    tile_end = pc // T

    tiota = lax.broadcasted_iota(jnp.int32, (NT, E), 0)
    te = jnp.sum((tile_end <= tiota).astype(jnp.int32), axis=1)
    laneiota = lax.broadcasted_iota(jnp.int32, (1, E), 1)
    max_e = jnp.max(jnp.where(counts > 0, laneiota, 0))
    te_ref[...] = jnp.minimum(te, max_e)[None, :]

    pos = jnp.sum(onehot * off_excl, axis=1) + rank
    pos_ref[...] = pos[None, :]


def _router(x, gate_w):
    return pl.pallas_call(
        _router_body,
        out_shape=(
            jax.ShapeDtypeStruct((S, E), jnp.float32),
            jax.ShapeDtypeStruct((1, S), jnp.int32),
            jax.ShapeDtypeStruct((1, NT), jnp.int32),
        ),
    )(x, gate_w)


def _mlp_body(te_ref, x_ref, wg_ref, wu_ref, wd_ref, out_ref):
    x = x_ref[...]
    g = lax.dot_general(x, wg_ref[0], (((1,), (1,)), ((), ())),
                        preferred_element_type=jnp.float32)
    u = lax.dot_general(x, wu_ref[0], (((1,), (1,)), ((), ())),
                        preferred_element_type=jnp.float32)
    act = g * jax.nn.sigmoid(g) * u
    out_ref[...] = lax.dot_general(act, wd_ref[0], (((1,), (1,)), ((), ())),
                                   preferred_element_type=jnp.float32)


def _moe_mlp(tile_expert, x_pad, w_gate, w_up, w_down):
    grid_spec = pltpu.PrefetchScalarGridSpec(
        num_scalar_prefetch=1,
        grid=(NT,),
        in_specs=[
            pl.BlockSpec((T, H), lambda i, te: (i, 0)),
            pl.BlockSpec((1, I, H), lambda i, te: (te[i], 0, 0)),
            pl.BlockSpec((1, I, H), lambda i, te: (te[i], 0, 0)),
            pl.BlockSpec((1, H, I), lambda i, te: (te[i], 0, 0)),
        ],
        out_specs=pl.BlockSpec((T, H), lambda i, te: (i, 0)),
    )
    return pl.pallas_call(
        _mlp_body,
        grid_spec=grid_spec,
        out_shape=jax.ShapeDtypeStruct((PAD, H), jnp.float32),
    )(tile_expert, x_pad, w_gate, w_up, w_down)


def _sc_wid():
    return lax.axis_index("s") * 2 + lax.axis_index("c")


@functools.cache
def _sc_kernels():
    mesh = plsc.VectorSubcoreMesh(
        core_axis_name="c", subcore_axis_name="s", num_cores=2, num_subcores=16)
    scratch = [
        pltpu.VMEM((RW,), jnp.int32),
        pltpu.VMEM((RW, H), jnp.float32),
        pltpu.SemaphoreType.DMA,
    ]

    @functools.partial(
        pl.kernel,
        out_type=jax.ShapeDtypeStruct((PAD, H), jnp.float32),
        mesh=mesh,
        scratch_types=scratch,
    )
    def sc_scatter(x_hbm, pos_hbm, out_hbm, idx_v, rows_v, sem):
        wid = _sc_wid()
        pltpu.sync_copy(pos_hbm.at[wid], idx_v)
        pltpu.sync_copy(x_hbm.at[pl.ds(wid * RW, RW)], rows_v)
        pltpu.async_copy(rows_v, out_hbm.at[idx_v], sem).wait()

    @functools.partial(
        pl.kernel,
        out_type=jax.ShapeDtypeStruct((S, H), jnp.float32),
        mesh=mesh,
        scratch_types=scratch,
    )
    def sc_gather(rows_hbm, pos_hbm, out_hbm, idx_v, rows_v, sem):
        wid = _sc_wid()
        pltpu.sync_copy(pos_hbm.at[wid], idx_v)
        pltpu.async_copy(rows_hbm.at[idx_v], rows_v, sem).wait()
        pltpu.sync_copy(rows_v, out_hbm.at[pl.ds(wid * RW, RW)])

    return sc_scatter, sc_gather


def kernel(hidden_states, gate_w, w_gate, w_up, w_down):
    b, s, h = hidden_states.shape
    x = hidden_states.reshape(s * b, h)
    sc_scatter, sc_gather = _sc_kernels()
    logits, pos, tile_expert = _router(x, gate_w)
    pos2d = pos.reshape(NW, RW)
    x_pad = sc_scatter(x, pos2d)
    out_pad = _moe_mlp(tile_expert.reshape(NT), x_pad, w_gate, w_up, w_down)
    final = sc_gather(out_pad, pos2d)
    return final.reshape(b, s, h), logits


# speedup vs baseline: 7.2060x; 1.2487x over previous
import functools

import jax
import jax.numpy as jnp
from jax import lax
from jax.experimental import pallas as pl
from jax.experimental.pallas import tpu as pltpu
from jax.experimental.pallas import tpu_sc as plsc

S = 2048
H = 1024
I = 768
E = 64
T = 64
NT = 96
PAD = NT * T

NW = 32
RW = S // NW


def _router_body(x_ref, gw_ref, logits_ref, pos_ref, te_ref):
    x = x_ref[...]
    gw = gw_ref[...]
    logits = lax.dot_general(
        x, gw, (((1,), (1,)), ((), ())),
        preferred_element_type=jnp.float32,
        precision=lax.Precision.DEFAULT,
    )
    logits_ref[...] = logits

    m = jnp.max(logits, axis=1, keepdims=True)
    eiota = lax.broadcasted_iota(jnp.int32, (S, E), 1)
    sel = jnp.min(jnp.where(logits == m, eiota, E), axis=1)

    onehot = (eiota == sel[:, None]).astype(jnp.int32)

    c = onehot
    k = 1
    while k < S:
        c = c + jnp.concatenate(
            [jnp.zeros((k, E), jnp.int32), c[: S - k, :]], axis=0)
        k *= 2

    rank = jnp.sum(onehot * c, axis=1) - 1
    counts = c[S - 1 : S, :]
    padded = ((counts + (T - 1)) // T) * T

    pc = padded
    k = 1
    while k < E:
        pc = pc + jnp.concatenate(
            [jnp.zeros((1, k), jnp.int32), pc[:, : E - k]], axis=1)
        k *= 2
    off_excl = pc - padded
    tile_end = pc // T

    tiota = lax.broadcasted_iota(jnp.int32, (NT, E), 0)
    te = jnp.sum((tile_end <= tiota).astype(jnp.int32), axis=1)
    laneiota = lax.broadcasted_iota(jnp.int32, (1, E), 1)
    max_e = jnp.max(jnp.where(counts > 0, laneiota, 0))
    te_ref[...] = jnp.minimum(te, max_e)[None, :]

    pos = jnp.sum(onehot * off_excl, axis=1) + rank
    pos_ref[...] = pos[None, :]


def _router(x, gate_w):
    return pl.pallas_call(
        _router_body,
        out_shape=(
            jax.ShapeDtypeStruct((S, E), jnp.float32),
            jax.ShapeDtypeStruct((1, S), jnp.int32),
            jax.ShapeDtypeStruct((1, NT), jnp.int32),
        ),
    )(x, gate_w)


def _mlp_body(te_ref, x_ref, wg_ref, wu_ref, wd_ref, out_ref):
    x = x_ref[...]
    g = lax.dot_general(x, wg_ref[0], (((1,), (1,)), ((), ())),
                        preferred_element_type=jnp.float32)
    u = lax.dot_general(x, wu_ref[0], (((1,), (1,)), ((), ())),
                        preferred_element_type=jnp.float32)
    act = g * jax.nn.sigmoid(g) * u
    out_ref[...] = lax.dot_general(act, wd_ref[0], (((1,), (1,)), ((), ())),
                                   preferred_element_type=jnp.float32)


def _moe_mlp(tile_expert, x_pad, w_gate, w_up, w_down):
    grid_spec = pltpu.PrefetchScalarGridSpec(
        num_scalar_prefetch=1,
        grid=(NT,),
        in_specs=[
            pl.BlockSpec((T, H), lambda i, te: (i, 0)),
            pl.BlockSpec((1, I, H), lambda i, te: (te[i], 0, 0)),
            pl.BlockSpec((1, I, H), lambda i, te: (te[i], 0, 0)),
            pl.BlockSpec((1, H, I), lambda i, te: (te[i], 0, 0)),
        ],
        out_specs=pl.BlockSpec((T, H), lambda i, te: (i, 0)),
    )
    return pl.pallas_call(
        _mlp_body,
        grid_spec=grid_spec,
        out_shape=jax.ShapeDtypeStruct((PAD, H), jnp.float32),
    )(tile_expert, x_pad, w_gate, w_up, w_down)


def _sc_wid():
    return lax.axis_index("s") * 2 + lax.axis_index("c")


@functools.cache
def _sc_kernels():
    mesh = plsc.VectorSubcoreMesh(
        core_axis_name="c", subcore_axis_name="s", num_cores=2, num_subcores=16)
    scratch = [
        pltpu.VMEM((RW,), jnp.int32),
        pltpu.VMEM((RW, H), jnp.float32),
        pltpu.SemaphoreType.DMA,
    ]

    @functools.partial(
        pl.kernel,
        out_type=jax.ShapeDtypeStruct((PAD, H), jnp.float32),
        mesh=mesh,
        scratch_types=scratch,
    )
    def sc_scatter(x_hbm, pos_hbm, out_hbm, idx_v, rows_v, sem):
        wid = _sc_wid()
        pltpu.sync_copy(pos_hbm.at[wid], idx_v)
        pltpu.sync_copy(x_hbm.at[pl.ds(wid * RW, RW)], rows_v)
        pltpu.async_copy(rows_v, out_hbm.at[idx_v], sem).wait()

    @functools.partial(
        pl.kernel,
        out_type=jax.ShapeDtypeStruct((S, H), jnp.float32),
        mesh=mesh,
        scratch_types=scratch,
    )
    def sc_gather(rows_hbm, pos_hbm, out_hbm, idx_v, rows_v, sem):
        wid = _sc_wid()
        pltpu.sync_copy(pos_hbm.at[wid], idx_v)
        pltpu.async_copy(rows_hbm.at[idx_v], rows_v, sem).wait()
        pltpu.sync_copy(rows_v, out_hbm.at[pl.ds(wid * RW, RW)])

    return sc_scatter, sc_gather


def kernel(hidden_states, gate_w, w_gate, w_up, w_down):
    b, s, h = hidden_states.shape
    x = hidden_states.reshape(s * b, h)
    sc_scatter, sc_gather = _sc_kernels()
    logits, pos, tile_expert = _router(x, gate_w)
    pos2d = pos.reshape(NW, RW)
    x_pad = sc_scatter(x, pos2d)
    out_pad = _moe_mlp(tile_expert.reshape(NT), x_pad, w_gate, w_up, w_down)
    final = sc_gather(out_pad, pos2d)
    return final.reshape(b, s, h), logits


# speedup vs baseline: 8.1128x; 1.1258x over previous
import functools

import jax
import jax.numpy as jnp
from jax import lax
from jax.experimental import pallas as pl
from jax.experimental.pallas import tpu as pltpu
from jax.experimental.pallas import tpu_sc as plsc

S = 2048
H = 1024
I = 768
E = 64
T = 64
NT = 96
PAD = NT * T

NW = 32
RW = S // NW


def _router_body(x_ref, gw_ref, logits_ref, pos_ref, te_ref, used_ref):
    x = x_ref[...]
    gw = gw_ref[...]
    logits = lax.dot_general(
        x, gw, (((1,), (1,)), ((), ())),
        preferred_element_type=jnp.float32,
        precision=lax.Precision.DEFAULT,
    )
    logits_ref[...] = logits

    m = jnp.max(logits, axis=1, keepdims=True)
    eiota = lax.broadcasted_iota(jnp.int32, (S, E), 1)
    sel = jnp.min(jnp.where(logits == m, eiota, E), axis=1)

    onehot = (eiota == sel[:, None]).astype(jnp.int32)

    c = onehot
    k = 1
    while k < S:
        c = c + jnp.concatenate(
            [jnp.zeros((k, E), jnp.int32), c[: S - k, :]], axis=0)
        k *= 2

    rank = jnp.sum(onehot * c, axis=1) - 1
    counts = c[S - 1 : S, :]
    padded = ((counts + (T - 1)) // T) * T

    pc = padded
    k = 1
    while k < E:
        pc = pc + jnp.concatenate(
            [jnp.zeros((1, k), jnp.int32), pc[:, : E - k]], axis=1)
        k *= 2
    off_excl = pc - padded
    tile_end = pc // T

    tiota = lax.broadcasted_iota(jnp.int32, (NT, E), 0)
    te = jnp.sum((tile_end <= tiota).astype(jnp.int32), axis=1)
    laneiota = lax.broadcasted_iota(jnp.int32, (1, E), 1)
    max_e = jnp.max(jnp.where(counts > 0, laneiota, 0))
    te_ref[...] = jnp.minimum(te, max_e)[None, :]
    total_tiles = pc[0, E - 1] // T
    used_ref[...] = (tiota[:, 0] < total_tiles).astype(jnp.int32)[None, :]

    pos = jnp.sum(onehot * off_excl, axis=1) + rank
    pos_ref[...] = pos[None, :]


def _router(x, gate_w):
    return pl.pallas_call(
        _router_body,
        out_shape=(
            jax.ShapeDtypeStruct((S, E), jnp.float32),
            jax.ShapeDtypeStruct((1, S), jnp.int32),
            jax.ShapeDtypeStruct((1, NT), jnp.int32),
            jax.ShapeDtypeStruct((1, NT), jnp.int32),
        ),
    )(x, gate_w)


def _mlp_body(te_ref, used_ref, x_ref, wg_ref, wu_ref, wd_ref, out_ref):
    @pl.when(used_ref[pl.program_id(0)] != 0)
    def _():
        x = x_ref[...]
        g = lax.dot_general(x, wg_ref[0], (((1,), (1,)), ((), ())),
                            preferred_element_type=jnp.float32)
        u = lax.dot_general(x, wu_ref[0], (((1,), (1,)), ((), ())),
                            preferred_element_type=jnp.float32)
        act = g * jax.nn.sigmoid(g) * u
        out_ref[...] = lax.dot_general(act, wd_ref[0], (((1,), (1,)), ((), ())),
                                       preferred_element_type=jnp.float32)


def _moe_mlp(tile_expert, used, x_pad, w_gate, w_up, w_down):
    grid_spec = pltpu.PrefetchScalarGridSpec(
        num_scalar_prefetch=2,
        grid=(NT,),
        in_specs=[
            pl.BlockSpec((T, H), lambda i, te, us: (i, 0)),
            pl.BlockSpec((1, I, H), lambda i, te, us: (te[i], 0, 0)),
            pl.BlockSpec((1, I, H), lambda i, te, us: (te[i], 0, 0)),
            pl.BlockSpec((1, H, I), lambda i, te, us: (te[i], 0, 0)),
        ],
        out_specs=pl.BlockSpec((T, H), lambda i, te, us: (i, 0)),
    )
    return pl.pallas_call(
        _mlp_body,
        grid_spec=grid_spec,
        out_shape=jax.ShapeDtypeStruct((PAD, H), jnp.float32),
    )(tile_expert, used, x_pad, w_gate, w_up, w_down)


def _sc_wid():
    return lax.axis_index("s") * 2 + lax.axis_index("c")


@functools.cache
def _sc_kernels():
    mesh = plsc.VectorSubcoreMesh(
        core_axis_name="c", subcore_axis_name="s", num_cores=2, num_subcores=16)
    scratch = [
        pltpu.VMEM((RW,), jnp.int32),
        pltpu.VMEM((RW, H), jnp.float32),
        pltpu.SemaphoreType.DMA,
    ]

    @functools.partial(
        pl.kernel,
        out_type=jax.ShapeDtypeStruct((PAD, H), jnp.float32),
        mesh=mesh,
        scratch_types=scratch,
    )
    def sc_scatter(x_hbm, pos_hbm, out_hbm, idx_v, rows_v, sem):
        wid = _sc_wid()
        pltpu.sync_copy(pos_hbm.at[wid], idx_v)
        pltpu.sync_copy(x_hbm.at[pl.ds(wid * RW, RW)], rows_v)
        pltpu.async_copy(rows_v, out_hbm.at[idx_v], sem).wait()

    @functools.partial(
        pl.kernel,
        out_type=jax.ShapeDtypeStruct((S, H), jnp.float32),
        mesh=mesh,
        scratch_types=scratch,
    )
    def sc_gather(rows_hbm, pos_hbm, out_hbm, idx_v, rows_v, sem):
        wid = _sc_wid()
        pltpu.sync_copy(pos_hbm.at[wid], idx_v)
        pltpu.async_copy(rows_hbm.at[idx_v], rows_v, sem).wait()
        pltpu.sync_copy(rows_v, out_hbm.at[pl.ds(wid * RW, RW)])

    return sc_scatter, sc_gather


def kernel(hidden_states, gate_w, w_gate, w_up, w_down):
    b, s, h = hidden_states.shape
    x = hidden_states.reshape(s * b, h)
    sc_scatter, sc_gather = _sc_kernels()
    logits, pos, tile_expert, used = _router(x, gate_w)
    pos2d = pos.reshape(NW, RW)
    x_pad = sc_scatter(x, pos2d)
    out_pad = _moe_mlp(tile_expert.reshape(NT), used.reshape(NT), x_pad,
                       w_gate, w_up, w_down)
    final = sc_gather(out_pad, pos2d)
    return final.reshape(b, s, h), logits


# speedup vs baseline: 8.8128x; 1.0863x over previous
import functools

import jax
import jax.numpy as jnp
from jax import lax
from jax.experimental import pallas as pl
from jax.experimental.pallas import tpu as pltpu
from jax.experimental.pallas import tpu_sc as plsc

S = 2048
H = 1024
I = 768
E = 64
T = 64
NT = 96
PAD = NT * T

NW = 32
RW = S // NW


def _router_body(x_ref, gw_ref, logits_ref, pos_ref, te_ref, used_ref):
    x = x_ref[...]
    gw = gw_ref[...]
    logits = lax.dot_general(
        x, gw, (((1,), (1,)), ((), ())),
        preferred_element_type=jnp.float32,
        precision=lax.Precision.DEFAULT,
    )
    logits_ref[...] = logits

    m = jnp.max(logits, axis=1, keepdims=True)
    eiota = lax.broadcasted_iota(jnp.int32, (S, E), 1)
    sel = jnp.min(jnp.where(logits == m, eiota, E), axis=1)

    onehot = (eiota == sel[:, None]).astype(jnp.int32)

    c = onehot
    k = 1
    while k < S:
        c = c + jnp.concatenate(
            [jnp.zeros((k, E), jnp.int32), c[: S - k, :]], axis=0)
        k *= 2

    rank = jnp.sum(onehot * c, axis=1) - 1
    counts = c[S - 1 : S, :]
    padded = ((counts + (T - 1)) // T) * T

    pc = padded
    k = 1
    while k < E:
        pc = pc + jnp.concatenate(
            [jnp.zeros((1, k), jnp.int32), pc[:, : E - k]], axis=1)
        k *= 2
    off_excl = pc - padded
    tile_end = pc // T

    tiota = lax.broadcasted_iota(jnp.int32, (NT, E), 0)
    te = jnp.sum((tile_end <= tiota).astype(jnp.int32), axis=1)
    laneiota = lax.broadcasted_iota(jnp.int32, (1, E), 1)
    max_e = jnp.max(jnp.where(counts > 0, laneiota, 0))
    te_ref[...] = jnp.minimum(te, max_e)[None, :]
    total_tiles = pc[0, E - 1] // T
    used_ref[...] = jnp.full((1, NT), total_tiles, jnp.int32)

    pos = jnp.sum(onehot * off_excl, axis=1) + rank
    pos_ref[...] = pos[None, :]


def _router(x, gate_w):
    return pl.pallas_call(
        _router_body,
        out_shape=(
            jax.ShapeDtypeStruct((S, E), jnp.float32),
            jax.ShapeDtypeStruct((1, S), jnp.int32),
            jax.ShapeDtypeStruct((1, NT), jnp.int32),
            jax.ShapeDtypeStruct((1, NT), jnp.int32),
        ),
    )(x, gate_w)


def _mlp_body(te_ref, x_ref, wg_ref, wu_ref, wd_ref, out_ref):
    x = x_ref[...]
    g = lax.dot_general(x, wg_ref[0], (((1,), (1,)), ((), ())),
                        preferred_element_type=jnp.float32)
    u = lax.dot_general(x, wu_ref[0], (((1,), (1,)), ((), ())),
                        preferred_element_type=jnp.float32)
    act = g * jax.nn.sigmoid(g) * u
    out_ref[...] = lax.dot_general(act, wd_ref[0], (((1,), (1,)), ((), ())),
                                   preferred_element_type=jnp.float32)


def _moe_mlp(tile_expert, num_tiles, x_pad, w_gate, w_up, w_down):
    grid_spec = pltpu.PrefetchScalarGridSpec(
        num_scalar_prefetch=1,
        grid=(num_tiles,),
        in_specs=[
            pl.BlockSpec((T, H), lambda i, te: (i, 0)),
            pl.BlockSpec((1, I, H), lambda i, te: (te[i], 0, 0)),
            pl.BlockSpec((1, I, H), lambda i, te: (te[i], 0, 0)),
            pl.BlockSpec((1, H, I), lambda i, te: (te[i], 0, 0)),
        ],
        out_specs=pl.BlockSpec((T, H), lambda i, te: (i, 0)),
    )
    return pl.pallas_call(
        _mlp_body,
        grid_spec=grid_spec,
        out_shape=jax.ShapeDtypeStruct((PAD, H), jnp.float32),
    )(tile_expert, x_pad, w_gate, w_up, w_down)


def _sc_wid():
    return lax.axis_index("s") * 2 + lax.axis_index("c")


@functools.cache
def _sc_kernels():
    mesh = plsc.VectorSubcoreMesh(
        core_axis_name="c", subcore_axis_name="s", num_cores=2, num_subcores=16)
    scratch = [
        pltpu.VMEM((RW,), jnp.int32),
        pltpu.VMEM((RW, H), jnp.float32),
        pltpu.SemaphoreType.DMA,
    ]

    @functools.partial(
        pl.kernel,
        out_type=jax.ShapeDtypeStruct((PAD, H), jnp.float32),
        mesh=mesh,
        scratch_types=scratch,
    )
    def sc_scatter(x_hbm, pos_hbm, out_hbm, idx_v, rows_v, sem):
        wid = _sc_wid()
        pltpu.sync_copy(pos_hbm.at[wid], idx_v)
        pltpu.sync_copy(x_hbm.at[pl.ds(wid * RW, RW)], rows_v)
        pltpu.async_copy(rows_v, out_hbm.at[idx_v], sem).wait()

    @functools.partial(
        pl.kernel,
        out_type=jax.ShapeDtypeStruct((S, H), jnp.float32),
        mesh=mesh,
        scratch_types=scratch,
    )
    def sc_gather(rows_hbm, pos_hbm, out_hbm, idx_v, rows_v, sem):
        wid = _sc_wid()
        pltpu.sync_copy(pos_hbm.at[wid], idx_v)
        pltpu.async_copy(rows_hbm.at[idx_v], rows_v, sem).wait()
        pltpu.sync_copy(rows_v, out_hbm.at[pl.ds(wid * RW, RW)])

    return sc_scatter, sc_gather


def kernel(hidden_states, gate_w, w_gate, w_up, w_down):
    b, s, h = hidden_states.shape
    x = hidden_states.reshape(s * b, h)
    sc_scatter, sc_gather = _sc_kernels()
    logits, pos, tile_expert, ntiles = _router(x, gate_w)
    pos2d = pos.reshape(NW, RW)
    x_pad = sc_scatter(x, pos2d)
    out_pad = _moe_mlp(tile_expert.reshape(NT), ntiles.reshape(NT)[0], x_pad,
                       w_gate, w_up, w_down)
    final = sc_gather(out_pad, pos2d)
    return final.reshape(b, s, h), logits
